# baseline (device time: 36074 ns/iter reference)
import jax
import jax.numpy as jnp
from jax import lax
from jax.experimental import pallas as pl
from jax.experimental.pallas import tpu as pltpu

N_DEV = 16
CAP = 12


def kernel(x, router_W, route_idx, expert_W):
    m, d = x.shape
    e_local, _, h = expert_W.shape
    blk = e_local * CAP
    n_slots = N_DEV * blk

    def body(x_ref, rW_ref, idx_ref, w_ref, out_ref,
             blocks_ref, send_sems, recv_sems):
        my = lax.axis_index("i")

        ridx = idx_ref[:, :]
        row = lax.broadcasted_iota(jnp.int32, (m, m), 0)
        col = lax.broadcasted_iota(jnp.int32, (m, m), 1)
        tri = (row >= col).astype(jnp.bfloat16)

        e_mine = my * e_local + lax.broadcasted_iota(jnp.int32, (m, e_local), 1)
        oh_mine = (ridx == e_mine).astype(jnp.bfloat16)
        cnt_mine = jnp.dot(tri, oh_mine, preferred_element_type=jnp.float32)
        cnt_tok = jnp.sum(cnt_mine * oh_mine.astype(jnp.float32), axis=1,
                          keepdims=True).astype(jnp.int32)
        slot_mine = (ridx - my * e_local) * CAP + cnt_tok - 1
        slot_mine = jnp.where(
            jnp.logical_and(cnt_tok >= 1, cnt_tok <= CAP), slot_mine, -1)
        gt = (slot_mine == lax.broadcasted_iota(jnp.int32, (m, blk), 1)
              ).astype(jnp.bfloat16)
        xb = x_ref[:, :].astype(jnp.bfloat16)
        xc = lax.dot_general(gt, xb, (((0,), (0,)), ((), ())),
                             preferred_element_type=jnp.float32)
        xc = xc.astype(jnp.bfloat16)
        parts = []
        for le in range(e_local):
            w = w_ref[le, :, :].astype(jnp.bfloat16)
            parts.append(jnp.dot(xc[le * CAP:(le + 1) * CAP, :], w,
                                 preferred_element_type=jnp.float32))
        block = jnp.concatenate(parts, axis=0).astype(jnp.bfloat16)
        blocks_ref[pl.ds(my * blk, blk), :] = block

        sends = []
        for k in range(1, N_DEV):
            j = (my + k) % N_DEV
            r = pltpu.make_async_remote_copy(
                src_ref=blocks_ref.at[pl.ds(my * blk, blk), :],
                dst_ref=blocks_ref.at[pl.ds(my * blk, blk), :],
                send_sem=send_sems.at[k],
                recv_sem=recv_sems.at[N_DEV - k],
                device_id=(j,),
                device_id_type=pl.DeviceIdType.MESH,
            )
            r.start()
            sends.append(r)

        e_all = lax.broadcasted_iota(jnp.int32, (m, N_DEV * e_local), 1)
        onehot = (ridx == e_all).astype(jnp.bfloat16)
        cnt = jnp.dot(tri, onehot, preferred_element_type=jnp.float32)
        cnt_all = jnp.sum(cnt * onehot.astype(jnp.float32), axis=1,
                          keepdims=True).astype(jnp.int32)
        slot = ridx * CAP + cnt_all - 1
        slot = jnp.where(cnt_all <= CAP, slot, -1)
        iota_blk = lax.broadcasted_iota(jnp.int32, (m, blk), 1)

        p_mine = (slot == iota_blk + my * blk).astype(jnp.bfloat16)
        acc = lax.dot_general(p_mine, block, (((1,), (0,)), ((), ())),
                              preferred_element_type=jnp.float32)

        for s in range(1, N_DEV):
            origin = (my + s) % N_DEV
            pltpu.make_async_remote_copy(
                src_ref=blocks_ref.at[pl.ds(0, blk), :],
                dst_ref=blocks_ref.at[pl.ds(origin * blk, blk), :],
                send_sem=send_sems.at[0],
                recv_sem=recv_sems.at[s],
                device_id=(my,),
                device_id_type=pl.DeviceIdType.MESH,
            ).wait_recv()
            p_o = (slot == iota_blk + origin * blk).astype(jnp.bfloat16)
            b_o = blocks_ref[pl.ds(origin * blk, blk), :]
            acc = acc + jnp.dot(p_o, b_o, preferred_element_type=jnp.float32)

        out_ref[:, :] = acc.astype(jnp.bfloat16)

        for r in sends:
            r.wait_send()

    return pl.pallas_call(
        body,
        out_shape=jax.ShapeDtypeStruct((m, h), jnp.bfloat16),
        in_specs=[pl.BlockSpec(memory_space=pltpu.VMEM)] * 4,
        out_specs=pl.BlockSpec(memory_space=pltpu.VMEM),
        scratch_shapes=[
            pltpu.VMEM((n_slots, h), jnp.bfloat16),
            pltpu.SemaphoreType.DMA((N_DEV,)),
            pltpu.SemaphoreType.DMA((N_DEV,)),
        ],
    )(x, router_W, route_idx, expert_W)


# device time: 29226 ns/iter; 1.2343x vs baseline; 1.2343x over previous
import jax
import jax.numpy as jnp
from jax import lax
from jax.experimental import pallas as pl
from jax.experimental.pallas import tpu as pltpu

N_DEV = 16
CAP = 12


def kernel(x, router_W, route_idx, expert_W):
    m, d = x.shape
    e_local, _, h = expert_W.shape
    blk = e_local * CAP
    n_slots = N_DEV * blk

    def body(x_ref, rW_ref, idx_ref, w_ref, out_ref,
             blocks_ref, send_sems, recv_sems):
        my = lax.axis_index("i")

        ridx = idx_ref[:, :]
        row = lax.broadcasted_iota(jnp.int32, (m, m), 0)
        col = lax.broadcasted_iota(jnp.int32, (m, m), 1)
        tri = (row >= col).astype(jnp.bfloat16)

        e_mine = my * e_local + lax.broadcasted_iota(jnp.int32, (m, e_local), 1)
        oh_mine = (ridx == e_mine).astype(jnp.bfloat16)
        cnt_mine = jnp.dot(tri, oh_mine, preferred_element_type=jnp.float32)
        cnt_tok = jnp.sum(cnt_mine * oh_mine.astype(jnp.float32), axis=1,
                          keepdims=True).astype(jnp.int32)
        slot_mine = (ridx - my * e_local) * CAP + cnt_tok - 1
        slot_mine = jnp.where(
            jnp.logical_and(cnt_tok >= 1, cnt_tok <= CAP), slot_mine, -1)
        gt = (slot_mine == lax.broadcasted_iota(jnp.int32, (m, blk), 1)
              ).astype(jnp.bfloat16)
        xb = x_ref[:, :].astype(jnp.bfloat16)
        xc = lax.dot_general(gt, xb, (((0,), (0,)), ((), ())),
                             preferred_element_type=jnp.float32)
        xc = xc.astype(jnp.bfloat16)
        parts = []
        for le in range(e_local):
            w = w_ref[le, :, :].astype(jnp.bfloat16)
            parts.append(jnp.dot(xc[le * CAP:(le + 1) * CAP, :], w,
                                 preferred_element_type=jnp.float32))
        block = jnp.concatenate(parts, axis=0).astype(jnp.bfloat16)
        blocks_ref[pl.ds(my * blk, blk), :] = block

        sends = []
        for k in range(1, N_DEV):
            j = (my + k) % N_DEV
            r = pltpu.make_async_remote_copy(
                src_ref=blocks_ref.at[pl.ds(my * blk, blk), :],
                dst_ref=blocks_ref.at[pl.ds(my * blk, blk), :],
                send_sem=send_sems.at[k],
                recv_sem=recv_sems.at[N_DEV - k],
                device_id=(j,),
                device_id_type=pl.DeviceIdType.MESH,
            )
            r.start()
            sends.append(r)

        e_all = lax.broadcasted_iota(jnp.int32, (m, N_DEV * e_local), 1)
        onehot = (ridx == e_all).astype(jnp.bfloat16)
        cnt = jnp.dot(tri, onehot, preferred_element_type=jnp.float32)
        cnt_all = jnp.sum(cnt * onehot.astype(jnp.float32), axis=1,
                          keepdims=True).astype(jnp.int32)
        slot = ridx * CAP + cnt_all - 1
        slot = jnp.where(cnt_all <= CAP, slot, -1)
        slot_all = lax.broadcasted_iota(jnp.int32, (m, n_slots), 1)
        p_mat = (slot == slot_all).astype(jnp.bfloat16)

        for s in range(1, N_DEV):
            origin = (my + s) % N_DEV
            pltpu.make_async_remote_copy(
                src_ref=blocks_ref.at[pl.ds(0, blk), :],
                dst_ref=blocks_ref.at[pl.ds(origin * blk, blk), :],
                send_sem=send_sems.at[0],
                recv_sem=recv_sems.at[s],
                device_id=(my,),
                device_id_type=pl.DeviceIdType.MESH,
            ).wait_recv()

        out_ref[:, :] = jnp.dot(
            p_mat, blocks_ref[:, :], preferred_element_type=jnp.float32
        ).astype(jnp.bfloat16)

        for r in sends:
            r.wait_send()

    return pl.pallas_call(
        body,
        out_shape=jax.ShapeDtypeStruct((m, h), jnp.bfloat16),
        in_specs=[pl.BlockSpec(memory_space=pltpu.VMEM)] * 4,
        out_specs=pl.BlockSpec(memory_space=pltpu.VMEM),
        scratch_shapes=[
            pltpu.VMEM((n_slots, h), jnp.bfloat16),
            pltpu.SemaphoreType.DMA((N_DEV,)),
            pltpu.SemaphoreType.DMA((N_DEV,)),
        ],
    )(x, router_W, route_idx, expert_W)


# device time: 26683 ns/iter; 1.3519x vs baseline; 1.0953x over previous
import jax
import jax.numpy as jnp
from jax import lax
from jax.experimental import pallas as pl
from jax.experimental.pallas import tpu as pltpu

N_DEV = 16
CAP = 12


def kernel(x, router_W, route_idx, expert_W):
    m, d = x.shape
    e_local, _, h = expert_W.shape
    n_slots = N_DEV * e_local * CAP
    blk = e_local * CAP

    def body(x_ref, rW_ref, idx_ref, w_ref, out_ref,
             blocks_ref, send_sems, recv_sems):
        my = lax.axis_index("i")

        ridx = idx_ref[:, :]
        e_all = lax.broadcasted_iota(jnp.int32, (m, N_DEV * e_local), 1)
        onehot = (ridx == e_all).astype(jnp.bfloat16)
        row = lax.broadcasted_iota(jnp.int32, (m, m), 0)
        col = lax.broadcasted_iota(jnp.int32, (m, m), 1)
        tri = (row >= col).astype(jnp.bfloat16)
        cnt = jnp.dot(tri, onehot, preferred_element_type=jnp.float32)
        cnt_tok = jnp.sum(cnt * onehot.astype(jnp.float32), axis=1,
                          keepdims=True).astype(jnp.int32)
        slot = ridx * CAP + cnt_tok - 1
        slot = jnp.where(cnt_tok <= CAP, slot, -1)

        slot_local = lax.broadcasted_iota(jnp.int32, (m, blk), 1) + my * blk
        gt = (slot == slot_local).astype(jnp.bfloat16)
        xb = x_ref[:, :].astype(jnp.bfloat16)
        xc = lax.dot_general(gt, xb, (((0,), (0,)), ((), ())),
                             preferred_element_type=jnp.float32)
        xc = xc.astype(jnp.bfloat16)
        parts = []
        for le in range(e_local):
            w = w_ref[le, :, :].astype(jnp.bfloat16)
            parts.append(jnp.dot(xc[le * CAP:(le + 1) * CAP, :], w,
                                 preferred_element_type=jnp.float32))
        block = jnp.concatenate(parts, axis=0).astype(jnp.bfloat16)
        blocks_ref[pl.ds(my * blk, blk), :] = block

        sends = []
        for k in range(1, N_DEV):
            j = (my + k) % N_DEV
            r = pltpu.make_async_remote_copy(
                src_ref=blocks_ref.at[pl.ds(my * blk, blk), :],
                dst_ref=blocks_ref.at[pl.ds(my * blk, blk), :],
                send_sem=send_sems.at[k],
                recv_sem=recv_sems.at[N_DEV - k],
                device_id=(j,),
                device_id_type=pl.DeviceIdType.MESH,
            )
            r.start()
            sends.append(r)

        slot_all = lax.broadcasted_iota(jnp.int32, (m, n_slots), 1)
        p_mat = (slot == slot_all).astype(jnp.bfloat16)

        for s in range(1, N_DEV):
            origin = (my + s) % N_DEV
            pltpu.make_async_remote_copy(
                src_ref=blocks_ref.at[pl.ds(0, blk), :],
                dst_ref=blocks_ref.at[pl.ds(origin * blk, blk), :],
                send_sem=send_sems.at[0],
                recv_sem=recv_sems.at[s],
                device_id=(my,),
                device_id_type=pl.DeviceIdType.MESH,
            ).wait_recv()

        out_ref[:, :] = jnp.dot(
            p_mat, blocks_ref[:, :], preferred_element_type=jnp.float32
        ).astype(jnp.bfloat16)

        for r in sends:
            r.wait_send()

    return pl.pallas_call(
        body,
        out_shape=jax.ShapeDtypeStruct((m, h), jnp.bfloat16),
        in_specs=[pl.BlockSpec(memory_space=pltpu.VMEM)] * 4,
        out_specs=pl.BlockSpec(memory_space=pltpu.VMEM),
        scratch_shapes=[
            pltpu.VMEM((n_slots, h), jnp.bfloat16),
            pltpu.SemaphoreType.DMA((N_DEV,)),
            pltpu.SemaphoreType.DMA((N_DEV,)),
        ],
    )(x, router_W, route_idx, expert_W)


# device time: 20313 ns/iter; 1.7759x vs baseline; 1.3136x over previous
import jax
import jax.numpy as jnp
from jax import lax
from jax.experimental import pallas as pl
from jax.experimental.pallas import tpu as pltpu

N_DEV = 16
CAP = 12


def kernel(x, router_W, route_idx, expert_W):
    m, d = x.shape
    e_local, _, h = expert_W.shape
    n_slots = N_DEV * e_local * CAP
    blk = e_local * CAP

    def body(x_ref, rW_ref, idx_ref, w_ref, out_ref,
             blocks_ref, send_sems, recv_sems):
        my = lax.axis_index("i")

        barrier_sem = pltpu.get_barrier_semaphore()
        for k in range(1, N_DEV):
            pl.semaphore_signal(
                barrier_sem, inc=1,
                device_id=((my + k) % N_DEV,),
                device_id_type=pl.DeviceIdType.MESH,
            )

        ridx = idx_ref[:, :]
        e_all = lax.broadcasted_iota(jnp.int32, (m, N_DEV * e_local), 1)
        onehot = (ridx == e_all).astype(jnp.bfloat16)
        mh = m // 2
        row = lax.broadcasted_iota(jnp.int32, (mh, mh), 0)
        col = lax.broadcasted_iota(jnp.int32, (mh, mh), 1)
        tri = (row >= col).astype(jnp.bfloat16)
        c_lo = jnp.dot(tri, onehot[:mh, :], preferred_element_type=jnp.float32)
        c_hi = jnp.dot(tri, onehot[mh:, :], preferred_element_type=jnp.float32)
        cnt = jnp.concatenate([c_lo, c_hi + c_lo[mh - 1:mh, :]], axis=0)
        cnt_tok = jnp.sum(cnt * onehot.astype(jnp.float32), axis=1,
                          keepdims=True).astype(jnp.int32)
        slot = ridx * CAP + cnt_tok - 1
        slot = jnp.where(cnt_tok <= CAP, slot, -1)

        slot_local = lax.broadcasted_iota(jnp.int32, (m, blk), 1) + my * blk
        gt = (slot == slot_local).astype(jnp.bfloat16)
        xb = x_ref[:, :].astype(jnp.bfloat16)
        xc = lax.dot_general(gt, xb, (((0,), (0,)), ((), ())),
                             preferred_element_type=jnp.float32)
        xc = xc.astype(jnp.bfloat16)
        parts = []
        for le in range(e_local):
            w = w_ref[le, :, :].astype(jnp.bfloat16)
            parts.append(jnp.dot(xc[le * CAP:(le + 1) * CAP, :], w,
                                 preferred_element_type=jnp.float32))
        block = jnp.concatenate(parts, axis=0).astype(jnp.bfloat16)
        blocks_ref[pl.ds(0, blk), :] = block

        pl.semaphore_wait(barrier_sem, N_DEV - 1)
        sends = []
        for k in range(1, N_DEV):
            j = (my + k) % N_DEV
            off = N_DEV - k
            r = pltpu.make_async_remote_copy(
                src_ref=blocks_ref.at[pl.ds(0, blk), :],
                dst_ref=blocks_ref.at[pl.ds(off * blk, blk), :],
                send_sem=send_sems.at[k],
                recv_sem=recv_sems.at[off],
                device_id=(j,),
                device_id_type=pl.DeviceIdType.MESH,
            )
            r.start()
            sends.append(r)

        col = lax.broadcasted_iota(jnp.int32, (m, n_slots), 1) + my * blk
        gslot = col - jnp.where(col >= n_slots, n_slots, 0)
        p_mat = (slot == gslot).astype(jnp.bfloat16)

        for s in range(1, N_DEV):
            pltpu.make_async_remote_copy(
                src_ref=blocks_ref.at[pl.ds(0, blk), :],
                dst_ref=blocks_ref.at[pl.ds(s * blk, blk), :],
                send_sem=send_sems.at[0],
                recv_sem=recv_sems.at[s],
                device_id=(my,),
                device_id_type=pl.DeviceIdType.MESH,
            ).wait_recv()

        out_ref[:, :] = jnp.dot(
            p_mat, blocks_ref[:, :], preferred_element_type=jnp.float32
        ).astype(jnp.bfloat16)

        for r in sends:
            r.wait_send()

    return pl.pallas_call(
        body,
        out_shape=jax.ShapeDtypeStruct((m, h), jnp.bfloat16),
        in_specs=[pl.BlockSpec(memory_space=pltpu.VMEM)] * 4,
        out_specs=pl.BlockSpec(memory_space=pltpu.VMEM),
        scratch_shapes=[
            pltpu.VMEM((n_slots, h), jnp.bfloat16),
            pltpu.SemaphoreType.DMA((N_DEV,)),
            pltpu.SemaphoreType.DMA((N_DEV,)),
        ],
        compiler_params=pltpu.CompilerParams(collective_id=0),
    )(x, router_W, route_idx, expert_W)
